# baseline (speedup 1.0000x reference)
import jax
import jax.numpy as jnp
from jax.experimental import pallas as pl

_H, _W = 50, 38
_PW = _W + 2
_N = 2048
_C = 512
_M = 64


def _fold_body(cw_ref, w2_ref, b1_ref, cb_ref, cww_ref, bias_ref):
    cww_ref[:] = jnp.dot(cw_ref[:], w2_ref[:],
                         preferred_element_type=jnp.float32)
    bias_ref[:] = jnp.dot(cw_ref[:], b1_ref[:],
                          preferred_element_type=jnp.float32) + cb_ref[:]


def _conv_body(f_ref, cww_ref, bias_ref, out_ref):
    acc = jnp.zeros((_M, _N), jnp.float32)
    for ky in range(3):
        for kx in range(3):
            t = ky * 3 + kx
            off = ky * _PW + kx
            acc = acc + jnp.dot(cww_ref[t], f_ref[:, off:off + _N],
                                preferred_element_type=jnp.float32)
    out_ref[:] = acc + bias_ref[:]


def kernel(out_map, conv1_w, conv1_b, loc_w, loc_b, score_w, score_b):
    x = out_map[0]
    xp = jnp.pad(x, ((0, 0), (1, 1), (1, 1)))
    f = xp.reshape(_C, (_H + 2) * _PW)
    f = jnp.pad(f, ((0, 0), (0, _N + 2 * _PW + 2 - f.shape[1])))
    f = f.astype(jnp.bfloat16)

    w2 = conv1_w.reshape(_C, _C * 9)
    cw = jnp.concatenate([loc_w[:, :, 0, 0], score_w[:, :, 0, 0]], axis=0)
    cw = jnp.pad(cw, ((0, _M - cw.shape[0]), (0, 0)))
    cb = jnp.pad(jnp.concatenate([loc_b, score_b]),
                 (0, _M - 54)).reshape(_M, 1)
    b1 = conv1_b.reshape(_C, 1)

    cww, bias = pl.pallas_call(
        _fold_body,
        out_shape=(jax.ShapeDtypeStruct((_M, _C * 9), jnp.float32),
                   jax.ShapeDtypeStruct((_M, 1), jnp.float32)),
    )(cw, w2, b1, cb)

    cww9 = cww.reshape(_M, _C, 9).transpose(2, 0, 1).astype(jnp.bfloat16)

    out = pl.pallas_call(
        _conv_body,
        out_shape=jax.ShapeDtypeStruct((_M, _N), jnp.float32),
    )(f, cww9, bias)

    out = out[:, :_H * _PW].reshape(_M, _H, _PW)[:, :, :_W]
    loc = out[:36][None]
    score = out[36:54][None]
    return (loc, score)
